# baseline (speedup 1.0000x reference)
import functools

import jax
import jax.numpy as jnp
from jax import lax
from jax.experimental import pallas as pl
from jax.experimental.pallas import tpu as pltpu
from jax.experimental.pallas import tpu_sc as plsc

NUM_VOCAB = 1000000
EMBDIM = 32
BATCH = 16384

_NC = 2
_NS = 16
_NW = _NC * _NS
_B_PER_W = BATCH // _NW
_CHUNK = 128
_NCHUNK = _B_PER_W // _CHUNK

_mesh = plsc.VectorSubcoreMesh(core_axis_name="c", subcore_axis_name="s")


@functools.partial(
    pl.kernel,
    mesh=_mesh,
    out_type=jax.ShapeDtypeStruct((BATCH, EMBDIM), jnp.float32),
    scratch_types=[
        pltpu.VMEM((_NCHUNK, _CHUNK), jnp.int32),
        pltpu.VMEM((_B_PER_W, EMBDIM), jnp.float32),
        pltpu.SemaphoreType.DMA,
    ],
    compiler_params=pltpu.CompilerParams(use_tc_tiling_on_sc=False),
)
def _emb_lookup(idx_hbm, table_hbm, out_hbm, idx_v, rows_v, sem):
    wid = lax.axis_index("s") * _NC + lax.axis_index("c")
    pltpu.sync_copy(idx_hbm.at[wid], idx_v)
    copies = []
    for j in range(_NCHUNK):
        copies.append(
            pltpu.async_copy(
                table_hbm.at[idx_v.at[j]],
                rows_v.at[pl.ds(j * _CHUNK, _CHUNK)],
                sem,
            )
        )
    for c in copies:
        c.wait()
    pltpu.sync_copy(rows_v, out_hbm.at[pl.ds(wid * _B_PER_W, _B_PER_W)])


def kernel(vocab_ids, table):
    idx = vocab_ids.astype(jnp.int32).reshape(_NW, _NCHUNK, _CHUNK)
    return _emb_lookup(idx, table)


# speedup vs baseline: 4.0977x; 4.0977x over previous
import functools

import jax
import jax.numpy as jnp
from jax import lax
from jax.experimental import pallas as pl
from jax.experimental.pallas import tpu as pltpu
from jax.experimental.pallas import tpu_sc as plsc

NUM_VOCAB = 1000000
EMBDIM = 32
BATCH = 16384

_NC = 2
_NS = 16
_NW = _NC * _NS
_B_PER_W = BATCH // _NW
_NCHUNK = 4
_CHUNK = _B_PER_W // _NCHUNK
_LANES = 16
_WAVE = 8
_NWAVES = _B_PER_W // _WAVE

_mesh = plsc.VectorSubcoreMesh(core_axis_name="c", subcore_axis_name="s")


@functools.partial(
    pl.kernel,
    mesh=_mesh,
    out_type=jax.ShapeDtypeStruct((EMBDIM, BATCH), jnp.float32),
    scratch_types=[
        pltpu.VMEM((_NCHUNK + 1, _CHUNK), jnp.int32),
        pltpu.VMEM((2, _WAVE, EMBDIM, 128), jnp.float32),
        pltpu.VMEM((EMBDIM, _B_PER_W), jnp.float32),
        pltpu.SemaphoreType.DMA((2, _WAVE)),
    ],
    compiler_params=pltpu.CompilerParams(needs_layout_passes=False),
)
def _emb_lookup(idx_hbm, tabT_hbm, outT_hbm, idx_v, banks, outv, sems):
    wid = lax.axis_index("s") * _NC + lax.axis_index("c")

    pltpu.sync_copy(idx_hbm.at[wid], idx_v.at[pl.ds(0, _NCHUNK)])

    row16 = lax.iota(jnp.int32, _LANES)

    def load_wave_vec(w):
        o = w * _WAVE
        return idx_v[o >> 7, pl.ds(o & 127, _LANES)]

    def fire_wave(v16, bank):
        for k in range(_WAVE):
            v = v16[k]
            col0 = pl.multiple_of((v >> 7) * 128, 128)
            pltpu.async_copy(
                tabT_hbm.at[:, pl.ds(col0, 128)],
                banks.at[bank, k],
                sems.at[bank, k],
            )

    v16_0 = load_wave_vec(0)
    fire_wave(v16_0, 0)

    def body(w, v16_cur):
        bank = lax.rem(w, 2)
        v16_next = load_wave_vec(w + 1)

        @pl.when(w + 1 < _NWAVES)
        def _():
            fire_wave(v16_next, 1 - bank)

        for k in range(_WAVE):
            v = v16_cur[k]
            pltpu.make_async_copy(
                tabT_hbm.at[:, pl.ds(0, 128)],
                banks.at[bank, k],
                sems.at[bank, k],
            ).wait()
            lanes16 = jnp.full((_LANES,), v & 127, dtype=jnp.int32)
            pos16 = jnp.full((_LANES,), w * _WAVE + k, dtype=jnp.int32)
            for h in range(EMBDIM // _LANES):
                rows = row16 + h * _LANES
                col = plsc.load_gather(banks.at[bank, k], [rows, lanes16])
                plsc.store_scatter(outv, [rows, pos16], col)
        return v16_next

    lax.fori_loop(0, _NWAVES, body, v16_0)

    pltpu.sync_copy(outv, outT_hbm.at[:, pl.ds(wid * _B_PER_W, _B_PER_W)])


def kernel(vocab_ids, table):
    idx = vocab_ids.astype(jnp.int32).reshape(_NW, _NCHUNK, _CHUNK)
    return _emb_lookup(idx, table.T).T


# speedup vs baseline: 4.1091x; 1.0028x over previous
import functools

import jax
import jax.numpy as jnp
from jax import lax
from jax.experimental import pallas as pl
from jax.experimental.pallas import tpu as pltpu
from jax.experimental.pallas import tpu_sc as plsc

NUM_VOCAB = 1000000
EMBDIM = 32
BATCH = 16384

_NC = 2
_NS = 16
_NW = _NC * _NS
_B_PER_W = BATCH // _NW
_NCHUNK = 4
_CHUNK = _B_PER_W // _NCHUNK
_LANES = 16
_WAVE = 8
_NWAVES = _B_PER_W // _WAVE

_mesh = plsc.VectorSubcoreMesh(core_axis_name="c", subcore_axis_name="s")


@functools.partial(
    pl.kernel,
    mesh=_mesh,
    out_type=jax.ShapeDtypeStruct((EMBDIM, BATCH), jnp.float32),
    scratch_types=[
        pltpu.VMEM((_NCHUNK + 1, _CHUNK), jnp.int32),
        pltpu.VMEM((3, _WAVE, EMBDIM, 128), jnp.float32),
        pltpu.VMEM((EMBDIM, _B_PER_W), jnp.float32),
        pltpu.SemaphoreType.DMA((3, _WAVE)),
    ],
    compiler_params=pltpu.CompilerParams(needs_layout_passes=False),
)
def _emb_lookup(idx_hbm, tabT_hbm, outT_hbm, idx_v, banks, outv, sems):
    wid = lax.axis_index("s") * _NC + lax.axis_index("c")

    pltpu.sync_copy(idx_hbm.at[wid], idx_v.at[pl.ds(0, _NCHUNK)])

    row16 = lax.iota(jnp.int32, _LANES)

    def load_wave_vec(w):
        o = w * _WAVE
        return idx_v[o >> 7, pl.ds(o & 127, _LANES)]

    def fire_wave(v16, bank):
        for k in range(_WAVE):
            v = v16[k]
            col0 = pl.multiple_of((v >> 7) * 128, 128)
            pltpu.async_copy(
                tabT_hbm.at[:, pl.ds(col0, 128)],
                banks.at[bank, k],
                sems.at[bank, k],
            )

    v16_0 = load_wave_vec(0)
    fire_wave(v16_0, 0)
    v16_1 = load_wave_vec(1)
    fire_wave(v16_1, 1)

    def body(w, carry):
        v16_cur, v16_nxt = carry
        bank = lax.rem(w, 3)
        v16_nxt2 = load_wave_vec(w + 2)

        @pl.when(w + 2 < _NWAVES)
        def _():
            fire_wave(v16_nxt2, lax.rem(w + 2, 3))

        for k in range(_WAVE):
            v = v16_cur[k]
            pltpu.make_async_copy(
                tabT_hbm.at[:, pl.ds(0, 128)],
                banks.at[bank, k],
                sems.at[bank, k],
            ).wait()
            lanes16 = jnp.full((_LANES,), v & 127, dtype=jnp.int32)
            pos16 = jnp.full((_LANES,), w * _WAVE + k, dtype=jnp.int32)
            for h in range(EMBDIM // _LANES):
                rows = row16 + h * _LANES
                col = plsc.load_gather(banks.at[bank, k], [rows, lanes16])
                plsc.store_scatter(outv, [rows, pos16], col)
        return (v16_nxt, v16_nxt2)

    lax.fori_loop(0, _NWAVES, body, (v16_0, v16_1))

    pltpu.sync_copy(outv, outT_hbm.at[:, pl.ds(wid * _B_PER_W, _B_PER_W)])


def kernel(vocab_ids, table):
    idx = vocab_ids.astype(jnp.int32).reshape(_NW, _NCHUNK, _CHUNK)
    return _emb_lookup(idx, table.T).T


# speedup vs baseline: 4.1390x; 1.0073x over previous
import functools

import jax
import jax.numpy as jnp
from jax import lax
from jax.experimental import pallas as pl
from jax.experimental.pallas import tpu as pltpu
from jax.experimental.pallas import tpu_sc as plsc

NUM_VOCAB = 1000000
EMBDIM = 32
BATCH = 16384

_NC = 2
_NS = 16
_NW = _NC * _NS
_B_PER_W = BATCH // _NW
_NCHUNK = 4
_CHUNK = _B_PER_W // _NCHUNK
_LANES = 16
_WAVE = 8
_NWAVES = _B_PER_W // _WAVE

_mesh = plsc.VectorSubcoreMesh(core_axis_name="c", subcore_axis_name="s")


@functools.partial(
    pl.kernel,
    mesh=_mesh,
    out_type=jax.ShapeDtypeStruct((EMBDIM, BATCH), jnp.float32),
    scratch_types=[
        pltpu.VMEM((_NCHUNK + 1, _CHUNK), jnp.int32),
        pltpu.VMEM((3, _WAVE, 4, 8, 128), jnp.float32),
        pltpu.VMEM((EMBDIM, _B_PER_W), jnp.float32),
        pltpu.SemaphoreType.DMA((3, _WAVE)),
    ],
    compiler_params=pltpu.CompilerParams(needs_layout_passes=False),
)
def _emb_lookup(idx_hbm, tabT_hbm, outT_hbm, idx_v, banks, outv, sems):
    wid = lax.axis_index("s") * _NC + lax.axis_index("c")
    tabT3 = tabT_hbm.reshape(4, 8, NUM_VOCAB)

    pltpu.sync_copy(idx_hbm.at[wid], idx_v.at[pl.ds(0, _NCHUNK)])

    row16 = lax.iota(jnp.int32, _LANES)

    def load_wave_vec(w):
        o = w * _WAVE
        return idx_v[o >> 7, pl.ds(o & 127, _LANES)]

    def fire_wave(v16, bank):
        for k in range(_WAVE):
            v = v16[k]
            col0 = pl.multiple_of((v >> 7) * 128, 128)
            for p in range(4):
                pltpu.async_copy(
                    tabT3.at[p, :, pl.ds(col0, 128)],
                    banks.at[bank, k, p],
                    sems.at[bank, k],
                )

    v16_0 = load_wave_vec(0)
    fire_wave(v16_0, 0)
    v16_1 = load_wave_vec(1)
    fire_wave(v16_1, 1)

    def body(w, carry):
        v16_cur, v16_nxt = carry
        bank = lax.rem(w, 3)
        v16_nxt2 = load_wave_vec(w + 2)

        @pl.when(w + 2 < _NWAVES)
        def _():
            fire_wave(v16_nxt2, lax.rem(w + 2, 3))

        for k in range(_WAVE):
            v = v16_cur[k]
            pltpu.make_async_copy(
                tabT3.at[:, :, pl.ds(0, 128)],
                banks.at[bank, k],
                sems.at[bank, k],
            ).wait()
            lanes16 = jnp.full((_LANES,), v & 127, dtype=jnp.int32)
            pos16 = jnp.full((_LANES,), w * _WAVE + k, dtype=jnp.int32)
            for h in range(EMBDIM // _LANES):
                rows = row16 + h * _LANES
                col = plsc.load_gather(
                    banks.at[bank, k], [rows >> 3, rows & 7, lanes16]
                )
                plsc.store_scatter(outv, [rows, pos16], col)
        return (v16_nxt, v16_nxt2)

    lax.fori_loop(0, _NWAVES, body, (v16_0, v16_1))

    pltpu.sync_copy(outv, outT_hbm.at[:, pl.ds(wid * _B_PER_W, _B_PER_W)])


def kernel(vocab_ids, table):
    idx = vocab_ids.astype(jnp.int32).reshape(_NW, _NCHUNK, _CHUNK)
    return _emb_lookup(idx, table.T).T
